# baseline (speedup 1.0000x reference)
import jax
import jax.numpy as jnp
from jax.experimental import pallas as pl

_B, _S, _D = 4, 4096, 2048
_E, _K, _R = 8, 2, 8
_ER = _E * _R
_SCALE = 16.0 / 8.0
_BLK = 1024


_CHUNK = 128


def _moe_block(x_ref, tw_ref, wcat_ref, ball_ref, out_ref):
    for c in range(_BLK // _CHUNK):
        rows = pl.ds(c * _CHUNK, _CHUNK)
        z = jnp.dot(x_ref[rows, :], wcat_ref[...],
                    preferred_element_type=jnp.float32)
        lrep = z[:, :_ER]
        h = z[:, _ER:]

        m1 = jnp.max(lrep, axis=1, keepdims=True)
        masked = jnp.where(lrep == m1, -jnp.inf, lrep)
        m2 = jnp.max(masked, axis=1, keepdims=True)
        p2 = jnp.exp(m2 - m1)
        rden = tw_ref[rows, :] / (1.0 + p2)
        w1 = rden
        w2 = p2 * rden
        grep = jnp.where(lrep >= m2, jnp.where(lrep == m1, w1, w2), 0.0)

        out_ref[rows, :] = jnp.dot((h * grep).astype(jnp.bfloat16),
                                   ball_ref[...],
                                   preferred_element_type=jnp.float32)


def kernel(x, type_weight, Wg, A, Bw):
    n = _B * _S
    xf = x.reshape(n, _D)
    twf = type_weight.reshape(n, 1)
    wg_rep = jnp.repeat(Wg, _R, axis=1)
    a_all = jnp.transpose(A, (1, 0, 2)).reshape(_D, _ER)
    w_cat = jnp.concatenate([wg_rep, a_all], axis=1)
    b_scaled = (Bw.reshape(_ER, _D) * _SCALE).astype(jnp.bfloat16)
    y = pl.pallas_call(
        _moe_block,
        grid=(n // _BLK,),
        in_specs=[
            pl.BlockSpec((_BLK, _D), lambda i: (i, 0)),
            pl.BlockSpec((_BLK, 1), lambda i: (i, 0)),
            pl.BlockSpec((_D, 2 * _ER), lambda i: (0, 0)),
            pl.BlockSpec((_ER, _D), lambda i: (0, 0)),
        ],
        out_specs=pl.BlockSpec((_BLK, _D), lambda i: (i, 0)),
        out_shape=jax.ShapeDtypeStruct((n, _D), jnp.float32),
    )(xf, twf, w_cat, b_scaled)
    return y.reshape(_B, _S, _D)


# speedup vs baseline: 1.1457x; 1.1457x over previous
import jax
import jax.numpy as jnp
from jax.experimental import pallas as pl
from jax.experimental.pallas import tpu as pltpu

_B, _S, _D = 4, 4096, 2048
_E, _K, _R = 8, 2, 8
_ER = _E * _R
_SCALE = 16.0 / 8.0
_BLK = 1024


_CHUNK = 256


def _moe_block(x_ref, tw_ref, wcat_ref, ball_ref, out_ref):
    for c in range(_BLK // _CHUNK):
        rows = pl.ds(c * _CHUNK, _CHUNK)
        z = jnp.dot(x_ref[rows, :], wcat_ref[...],
                    preferred_element_type=jnp.float32)
        lrep = z[:, :_ER]
        h = z[:, _ER:]

        m1 = jnp.max(lrep, axis=1, keepdims=True)
        masked = jnp.where(lrep == m1, -jnp.inf, lrep)
        m2 = jnp.max(masked, axis=1, keepdims=True)
        p2 = jnp.exp(m2 - m1)
        rden = tw_ref[rows, :] / (1.0 + p2)
        w1 = rden
        w2 = p2 * rden
        grep = jnp.where(lrep >= m2, jnp.where(lrep == m1, w1, w2), 0.0)

        out_ref[rows, :] = jnp.dot((h * grep).astype(jnp.bfloat16),
                                   ball_ref[...],
                                   preferred_element_type=jnp.float32)


def kernel(x, type_weight, Wg, A, Bw):
    n = _B * _S
    xf = x.reshape(n, _D)
    twf = type_weight.reshape(n, 1)
    wg_rep = jnp.repeat(Wg, _R, axis=1)
    a_all = jnp.transpose(A, (1, 0, 2)).reshape(_D, _ER)
    w_cat = jnp.concatenate([wg_rep, a_all], axis=1)
    b_scaled = (Bw.reshape(_ER, _D) * _SCALE).astype(jnp.bfloat16)
    y = pl.pallas_call(
        _moe_block,
        grid=(n // _BLK,),
        in_specs=[
            pl.BlockSpec((_BLK, _D), lambda i: (i, 0)),
            pl.BlockSpec((_BLK, 1), lambda i: (i, 0)),
            pl.BlockSpec((_D, 2 * _ER), lambda i: (0, 0)),
            pl.BlockSpec((_ER, _D), lambda i: (0, 0)),
        ],
        out_specs=pl.BlockSpec((_BLK, _D), lambda i: (i, 0)),
        out_shape=jax.ShapeDtypeStruct((n, _D), jnp.float32),
        compiler_params=pltpu.CompilerParams(
            dimension_semantics=("parallel",)),
    )(xf, twf, w_cat, b_scaled)
    return y.reshape(_B, _S, _D)


# speedup vs baseline: 1.2004x; 1.0477x over previous
import jax
import jax.numpy as jnp
from jax.experimental import pallas as pl
from jax.experimental.pallas import tpu as pltpu

_B, _S, _D = 4, 4096, 2048
_E, _K, _R = 8, 2, 8
_ER = _E * _R
_SCALE = 16.0 / 8.0
_BLK = 1024


_CHUNK = 256


def _moe_block(x_ref, tw_ref, wcat_ref, ball_ref, out_ref):
    for c in range(_BLK // _CHUNK):
        rows = pl.ds(c * _CHUNK, _CHUNK)
        z = jnp.dot(x_ref[rows, :], wcat_ref[...],
                    preferred_element_type=jnp.float32)
        lrep = z[:, :_ER]
        h = z[:, _ER:]

        m1 = jnp.max(lrep, axis=1, keepdims=True)
        masked = jnp.where(lrep == m1, -jnp.inf, lrep)
        m2 = jnp.max(masked, axis=1, keepdims=True)
        p2 = jnp.exp(m2 - m1)
        twc = jnp.transpose(tw_ref[:, rows], (1, 0))
        rden = twc / (1.0 + p2)
        w1 = rden
        w2 = p2 * rden
        grep = jnp.where(lrep >= m2, jnp.where(lrep == m1, w1, w2), 0.0)

        out_ref[rows, :] = jnp.dot((h * grep).astype(jnp.bfloat16),
                                   ball_ref[...],
                                   preferred_element_type=jnp.float32)


def kernel(x, type_weight, Wg, A, Bw):
    n = _B * _S
    xf = x.reshape(n, _D)
    twf = type_weight.reshape(1, n)
    wg_rep = jnp.repeat(Wg, _R, axis=1)
    a_all = jnp.transpose(A, (1, 0, 2)).reshape(_D, _ER)
    w_cat = jnp.concatenate([wg_rep, a_all], axis=1)
    b_scaled = (Bw.reshape(_ER, _D) * _SCALE).astype(jnp.bfloat16)
    y = pl.pallas_call(
        _moe_block,
        grid=(n // _BLK,),
        in_specs=[
            pl.BlockSpec((_BLK, _D), lambda i: (i, 0)),
            pl.BlockSpec((1, _BLK), lambda i: (0, i)),
            pl.BlockSpec((_D, 2 * _ER), lambda i: (0, 0)),
            pl.BlockSpec((_ER, _D), lambda i: (0, 0)),
        ],
        out_specs=pl.BlockSpec((_BLK, _D), lambda i: (i, 0)),
        out_shape=jax.ShapeDtypeStruct((n, _D), jnp.float32),
        compiler_params=pltpu.CompilerParams(
            dimension_semantics=("parallel",), vmem_limit_bytes=100*1024*1024),
    )(xf, twf, w_cat, b_scaled)
    return y.reshape(_B, _S, _D)
